# baseline (speedup 1.0000x reference)
import functools

import jax
import jax.numpy as jnp
from jax import lax
from jax.experimental import pallas as pl
from jax.experimental.pallas import tpu as pltpu
from jax.experimental.pallas import tpu_sc as plsc

N = 10000
E = 320000
D = 128
H = 64
L = 4
G = 64

N2 = N // 2
H2 = 2 * H

NC = 2
NS = 16
NW = NC * NS
EPW = E // NW
K = 125
KP = 128
NCH = EPW // K
NTRASH = 64
RPS = 624
RTAIL = N - NS * RPS
S = 8


def _sc_agg_body(h_hbm, src_hbm, dst_hbm, zeros_hbm, out_hbm,
                 src_v, dst_v, rows_v, *sems):
    agg_sh = sems[-1]
    gsem = sems[:S]
    ssem = sems[S:2 * S]
    cid = lax.axis_index("c")
    sid = lax.axis_index("s")
    wid = sid * NC + cid

    pltpu.sync_copy(zeros_hbm.at[pl.ds(sid * RPS, RPS)],
                    agg_sh.at[pl.ds(sid * RPS, RPS)])

    @pl.when(sid == 0)
    def _():
        pltpu.sync_copy(zeros_hbm.at[pl.ds(NS * RPS, RTAIL)],
                        agg_sh.at[pl.ds(NS * RPS, RTAIL)])

    pltpu.sync_copy(src_hbm.at[wid], src_v)
    pltpu.sync_copy(dst_hbm.at[wid], dst_v)
    plsc.subcore_barrier()

    for r in range(S):
        pltpu.async_copy(h_hbm.at[src_v.at[r]], rows_v.at[r], gsem[r])

    def body(i, carry):
        for r in range(S):
            j = i * S + r
            pltpu.make_async_copy(h_hbm.at[src_v.at[j]], rows_v.at[r],
                                  gsem[r]).wait()
            pltpu.async_copy(rows_v.at[r], agg_sh.at[dst_v.at[j]], ssem[r],
                             add=True)
        for r in range(S):
            j = i * S + r

            @pl.when(j + S < NCH)
            def _():
                pltpu.make_async_copy(rows_v.at[r], agg_sh.at[dst_v.at[j]],
                                      ssem[r]).wait()
                pltpu.async_copy(h_hbm.at[src_v.at[j + S]], rows_v.at[r],
                                gsem[r])
        return carry

    lax.fori_loop(0, NCH // S, body, 0)
    for r in range(S):
        j = NCH - S + r
        pltpu.make_async_copy(rows_v.at[r], agg_sh.at[dst_v.at[j]],
                              ssem[r]).wait()
    plsc.subcore_barrier()
    pltpu.sync_copy(agg_sh.at[pl.ds(sid * RPS, RPS)],
                    out_hbm.at[cid, pl.ds(sid * RPS, RPS)])

    @pl.when(sid == 0)
    def _():
        pltpu.sync_copy(agg_sh.at[pl.ds(NS * RPS, RTAIL)],
                        out_hbm.at[cid, pl.ds(NS * RPS, RTAIL)])


_sc_agg = functools.partial(
    pl.kernel,
    mesh=plsc.VectorSubcoreMesh(core_axis_name="c", subcore_axis_name="s",
                                num_cores=NC),
    compiler_params=pltpu.CompilerParams(use_tc_tiling_on_sc=False),
    out_type=jax.ShapeDtypeStruct((NC, N, H), jnp.float32),
    scratch_types=(
        [pltpu.VMEM((NCH, KP), jnp.int32),
         pltpu.VMEM((NCH, KP), jnp.int32),
         pltpu.VMEM((S, KP, H), jnp.float32)]
        + [pltpu.SemaphoreType.DMA] * (2 * S)
        + [pltpu.VMEM_SHARED((N + NTRASH, H), jnp.float32)]
    ),
)(_sc_agg_body)


def _embed_body(xe_ref, xo_ref, w_ref, b_ref, o_ref):
    he = jnp.dot(xe_ref[...], w_ref[...],
                 preferred_element_type=jnp.float32) + b_ref[...]
    ho = jnp.dot(xo_ref[...], w_ref[...],
                 preferred_element_type=jnp.float32) + b_ref[...]
    o_ref[...] = jnp.concatenate([he, ho], axis=1)


def _layer_body(h_ref, agg_ref, w1_ref, b1_ref, w2_ref, b2_ref,
                gm_ref, bt_ref, o_ref):
    z = h_ref[...] + agg_ref[0] + agg_ref[1]
    t = jnp.maximum(jnp.dot(z, w1_ref[...],
                            preferred_element_type=jnp.float32) + b1_ref[...],
                    0.0)
    z2 = (jnp.dot(t, w2_ref[...], preferred_element_type=jnp.float32)
          + b2_ref[...])
    s128 = jnp.mean(z2, axis=0, keepdims=True)
    m64 = 0.5 * (s128[:, :H] + s128[:, H:])
    mc = jnp.concatenate([m64, m64], axis=1)
    d = z2 - mc
    v128 = jnp.mean(d * d, axis=0, keepdims=True)
    v64 = 0.5 * (v128[:, :H] + v128[:, H:])
    vc = jnp.concatenate([v64, v64], axis=1)
    zn = d * lax.rsqrt(vc + 1e-5) * gm_ref[...] + bt_ref[...]
    o_ref[...] = jnp.maximum(zn, 0.0)


def _tail_body(h_ref, agg_ref, w1_ref, b1_ref, w2_ref, b2_ref, gm_ref,
               bt_ref, be_ref, bo_ref, wf1_ref, bf1_ref, wf2_ref, bf2_ref,
               o_ref):
    z = h_ref[...] + agg_ref[0] + agg_ref[1]
    t = jnp.maximum(jnp.dot(z, w1_ref[...],
                            preferred_element_type=jnp.float32) + b1_ref[...],
                    0.0)
    z2 = (jnp.dot(t, w2_ref[...], preferred_element_type=jnp.float32)
          + b2_ref[...])
    s128 = jnp.mean(z2, axis=0, keepdims=True)
    m64 = 0.5 * (s128[:, :H] + s128[:, H:])
    mc = jnp.concatenate([m64, m64], axis=1)
    d = z2 - mc
    v128 = jnp.mean(d * d, axis=0, keepdims=True)
    v64 = 0.5 * (v128[:, :H] + v128[:, H:])
    vc = jnp.concatenate([v64, v64], axis=1)
    zn = d * lax.rsqrt(vc + 1e-5) * gm_ref[...] + bt_ref[...]
    hh = jnp.maximum(zn, 0.0)
    ids = lax.broadcasted_iota(jnp.int32, (G, N2), 0)
    me = (ids == be_ref[...]).astype(jnp.float32)
    mo = (ids == bo_ref[...]).astype(jnp.float32)
    g = (jnp.dot(me, hh[:, :H], preferred_element_type=jnp.float32)
         + jnp.dot(mo, hh[:, H:], preferred_element_type=jnp.float32))
    r = jnp.maximum(jnp.dot(g, wf1_ref[...],
                            preferred_element_type=jnp.float32) + bf1_ref[...],
                    0.0)
    o_ref[...] = (jnp.dot(r, wf2_ref[...], preferred_element_type=jnp.float32)
                  + bf2_ref[...])


def _blockdiag(w):
    za = jnp.zeros_like(w)
    top = jnp.concatenate([w, za], axis=-1)
    bot = jnp.concatenate([za, w], axis=-1)
    return jnp.concatenate([top, bot], axis=-2)


def kernel(x, edge_index, batch, W_embed, b_embed, W1, b1, W2, b2,
           gamma, beta, W_fc1, b_fc1, W_fc2, b_fc2):
    src3 = edge_index[0].reshape(NW, NCH, K)
    dst3 = edge_index[1].reshape(NW, NCH, K)
    srcp = jnp.concatenate(
        [src3, jnp.zeros((NW, NCH, KP - K), jnp.int32)], axis=-1)
    w_ix = jnp.arange(NW, dtype=jnp.int32).reshape(NW, 1, 1)
    j_ix = jnp.arange(NCH, dtype=jnp.int32).reshape(1, NCH, 1)
    t_ix = jnp.arange(KP - K, dtype=jnp.int32).reshape(1, 1, KP - K)
    pad_dst = N + (w_ix * 17 + j_ix * 3 + t_ix) % NTRASH
    dstp = jnp.concatenate(
        [dst3, jnp.broadcast_to(pad_dst, (NW, NCH, KP - K))], axis=-1)
    zeros = jnp.zeros((N, H), jnp.float32)

    W1d = _blockdiag(W1)
    W2d = _blockdiag(W2)
    b1d = jnp.tile(b1, (1, 2))
    b2d = jnp.tile(b2, (1, 2))
    gmd = jnp.tile(gamma, (1, 2))
    btd = jnp.tile(beta, (1, 2))

    h2 = pl.pallas_call(
        _embed_body,
        out_shape=jax.ShapeDtypeStruct((N2, H2), jnp.float32),
    )(x[0::2], x[1::2], W_embed, b_embed.reshape(1, H))

    layer = pl.pallas_call(
        _layer_body,
        out_shape=jax.ShapeDtypeStruct((N2, H2), jnp.float32),
    )
    for l in range(L - 1):
        agg = _sc_agg(h2.reshape(N, H), srcp, dstp, zeros)
        h2 = layer(h2, agg.reshape(NC, N2, H2), W1d[l],
                   b1d[l].reshape(1, H2), W2d[l], b2d[l].reshape(1, H2),
                   gmd[l].reshape(1, H2), btd[l].reshape(1, H2))

    agg = _sc_agg(h2.reshape(N, H), srcp, dstp, zeros)
    bp = batch.reshape(N2, 2)
    l = L - 1
    out = pl.pallas_call(
        _tail_body,
        out_shape=jax.ShapeDtypeStruct((G, 1), jnp.float32),
    )(h2, agg.reshape(NC, N2, H2), W1d[l], b1d[l].reshape(1, H2), W2d[l],
      b2d[l].reshape(1, H2), gmd[l].reshape(1, H2), btd[l].reshape(1, H2),
      bp[:, 0].reshape(1, N2), bp[:, 1].reshape(1, N2), W_fc1,
      b_fc1.reshape(1, H), W_fc2, b_fc2.reshape(1, 1))
    return out.reshape(-1)


# speedup vs baseline: 2.8289x; 2.8289x over previous
import functools

import jax
import jax.numpy as jnp
from jax import lax
from jax.experimental import pallas as pl
from jax.experimental.pallas import tpu as pltpu
from jax.experimental.pallas import tpu_sc as plsc

N = 10000
E = 320000
D = 128
H = 64
L = 4
G = 64

N2 = N // 2
H2 = 2 * H

NC = 2
NS = 16
NW = NC * NS
EPW = E // NW
K = 125
NCH = EPW // K
RPS = 624
RTAIL = N - NS * RPS
S = 8


def _sc_agg_body(h_hbm, src_hbm, dst_hbm, zeros_hbm, out_hbm,
                 src_v, dst_v, rows_v, *sems):
    agg_sh = sems[-1]
    gsem = sems[:S]
    ssem = sems[S:2 * S]
    cid = lax.axis_index("c")
    sid = lax.axis_index("s")
    wid = sid * NC + cid

    pltpu.sync_copy(zeros_hbm.at[pl.ds(sid * RPS, RPS)],
                    agg_sh.at[pl.ds(sid * RPS, RPS)])

    @pl.when(sid == 0)
    def _():
        pltpu.sync_copy(zeros_hbm.at[pl.ds(NS * RPS, RTAIL)],
                        agg_sh.at[pl.ds(NS * RPS, RTAIL)])

    pltpu.sync_copy(src_hbm.at[wid], src_v)
    pltpu.sync_copy(dst_hbm.at[wid], dst_v)
    plsc.subcore_barrier()

    for r in range(S):
        pltpu.async_copy(h_hbm.at[src_v.at[r]], rows_v.at[r], gsem[r])

    def body(i, carry):
        for r in range(S):
            j = i * S + r
            pltpu.make_async_copy(h_hbm.at[src_v.at[j]], rows_v.at[r],
                                  gsem[r]).wait()
            pltpu.async_copy(rows_v.at[r], agg_sh.at[dst_v.at[j]], ssem[r],
                             add=True)
        for r in range(S):
            j = i * S + r

            @pl.when(j + S < NCH)
            def _():
                pltpu.make_async_copy(rows_v.at[r], agg_sh.at[dst_v.at[j]],
                                      ssem[r]).wait()
                pltpu.async_copy(h_hbm.at[src_v.at[j + S]], rows_v.at[r],
                                gsem[r])
        return carry

    lax.fori_loop(0, NCH // S, body, 0)
    for r in range(S):
        j = NCH - S + r
        pltpu.make_async_copy(rows_v.at[r], agg_sh.at[dst_v.at[j]],
                              ssem[r]).wait()
    plsc.subcore_barrier()
    pltpu.sync_copy(agg_sh.at[pl.ds(sid * RPS, RPS)],
                    out_hbm.at[cid, pl.ds(sid * RPS, RPS)])

    @pl.when(sid == 0)
    def _():
        pltpu.sync_copy(agg_sh.at[pl.ds(NS * RPS, RTAIL)],
                        out_hbm.at[cid, pl.ds(NS * RPS, RTAIL)])


_sc_agg = functools.partial(
    pl.kernel,
    mesh=plsc.VectorSubcoreMesh(core_axis_name="c", subcore_axis_name="s",
                                num_cores=NC),
    compiler_params=pltpu.CompilerParams(use_tc_tiling_on_sc=False),
    out_type=jax.ShapeDtypeStruct((NC, N, H), jnp.float32),
    scratch_types=(
        [pltpu.VMEM((NCH, K), jnp.int32),
         pltpu.VMEM((NCH, K), jnp.int32),
         pltpu.VMEM((S, K, H), jnp.float32)]
        + [pltpu.SemaphoreType.DMA] * (2 * S)
        + [pltpu.VMEM_SHARED((N, H), jnp.float32)]
    ),
)(_sc_agg_body)


def _embed_body(x_ref, w_ref, b_ref, o_ref):
    o_ref[...] = (jnp.dot(x_ref[...], w_ref[...],
                          preferred_element_type=jnp.float32) + b_ref[...])


def _layer_body(h_ref, agg_ref, w1_ref, b1_ref, w2_ref, b2_ref,
                gm_ref, bt_ref, o_ref):
    z = h_ref[...] + agg_ref[0] + agg_ref[1]
    t = jnp.maximum(jnp.dot(z, w1_ref[...],
                            preferred_element_type=jnp.float32) + b1_ref[...],
                    0.0)
    z2 = (jnp.dot(t, w2_ref[...], preferred_element_type=jnp.float32)
          + b2_ref[...])
    s128 = jnp.mean(z2, axis=0, keepdims=True)
    m64 = 0.5 * (s128[:, :H] + s128[:, H:])
    mc = jnp.concatenate([m64, m64], axis=1)
    d = z2 - mc
    v128 = jnp.mean(d * d, axis=0, keepdims=True)
    v64 = 0.5 * (v128[:, :H] + v128[:, H:])
    vc = jnp.concatenate([v64, v64], axis=1)
    zn = d * lax.rsqrt(vc + 1e-5) * gm_ref[...] + bt_ref[...]
    o_ref[...] = jnp.maximum(zn, 0.0)


def _tail_body(h_ref, agg_ref, w1_ref, b1_ref, w2_ref, b2_ref, gm_ref,
               bt_ref, be_ref, bo_ref, wf1_ref, bf1_ref, wf2_ref, bf2_ref,
               o_ref):
    z = h_ref[...] + agg_ref[0] + agg_ref[1]
    t = jnp.maximum(jnp.dot(z, w1_ref[...],
                            preferred_element_type=jnp.float32) + b1_ref[...],
                    0.0)
    z2 = (jnp.dot(t, w2_ref[...], preferred_element_type=jnp.float32)
          + b2_ref[...])
    s128 = jnp.mean(z2, axis=0, keepdims=True)
    m64 = 0.5 * (s128[:, :H] + s128[:, H:])
    mc = jnp.concatenate([m64, m64], axis=1)
    d = z2 - mc
    v128 = jnp.mean(d * d, axis=0, keepdims=True)
    v64 = 0.5 * (v128[:, :H] + v128[:, H:])
    vc = jnp.concatenate([v64, v64], axis=1)
    zn = d * lax.rsqrt(vc + 1e-5) * gm_ref[...] + bt_ref[...]
    hh = jnp.maximum(zn, 0.0)
    ids = lax.broadcasted_iota(jnp.int32, (G, N2), 0)
    me = (ids == be_ref[...]).astype(jnp.float32)
    mo = (ids == bo_ref[...]).astype(jnp.float32)
    g = (jnp.dot(me, hh[:, :H], preferred_element_type=jnp.float32)
         + jnp.dot(mo, hh[:, H:], preferred_element_type=jnp.float32))
    r = jnp.maximum(jnp.dot(g, wf1_ref[...],
                            preferred_element_type=jnp.float32) + bf1_ref[...],
                    0.0)
    o_ref[...] = (jnp.dot(r, wf2_ref[...], preferred_element_type=jnp.float32)
                  + bf2_ref[...])


def _blockdiag(w):
    za = jnp.zeros_like(w)
    top = jnp.concatenate([w, za], axis=-1)
    bot = jnp.concatenate([za, w], axis=-1)
    return jnp.concatenate([top, bot], axis=-2)


def kernel(x, edge_index, batch, W_embed, b_embed, W1, b1, W2, b2,
           gamma, beta, W_fc1, b_fc1, W_fc2, b_fc2):
    src = edge_index[0].reshape(NW, NCH, K)
    dst = edge_index[1].reshape(NW, NCH, K)
    zeros = jnp.zeros((N, H), jnp.float32)

    W1d = _blockdiag(W1)
    W2d = _blockdiag(W2)
    b1d = jnp.tile(b1, (1, 2))
    b2d = jnp.tile(b2, (1, 2))
    gmd = jnp.tile(gamma, (1, 2))
    btd = jnp.tile(beta, (1, 2))

    h2 = pl.pallas_call(
        _embed_body,
        out_shape=jax.ShapeDtypeStruct((N2, H2), jnp.float32),
    )(x.reshape(N2, 2 * D), _blockdiag(W_embed),
      jnp.tile(b_embed, 2).reshape(1, H2))

    layer = pl.pallas_call(
        _layer_body,
        out_shape=jax.ShapeDtypeStruct((N2, H2), jnp.float32),
    )
    for l in range(L - 1):
        agg = _sc_agg(h2.reshape(N, H), src, dst, zeros)
        h2 = layer(h2, agg.reshape(NC, N2, H2), W1d[l],
                   b1d[l].reshape(1, H2), W2d[l], b2d[l].reshape(1, H2),
                   gmd[l].reshape(1, H2), btd[l].reshape(1, H2))

    agg = _sc_agg(h2.reshape(N, H), src, dst, zeros)
    bp = batch.reshape(N2, 2)
    l = L - 1
    out = pl.pallas_call(
        _tail_body,
        out_shape=jax.ShapeDtypeStruct((G, 1), jnp.float32),
    )(h2, agg.reshape(NC, N2, H2), W1d[l], b1d[l].reshape(1, H2), W2d[l],
      b2d[l].reshape(1, H2), gmd[l].reshape(1, H2), btd[l].reshape(1, H2),
      bp[:, 0].reshape(1, N2), bp[:, 1].reshape(1, N2), W_fc1,
      b_fc1.reshape(1, H), W_fc2, b_fc2.reshape(1, 1))
    return out.reshape(-1)
